# baseline (speedup 1.0000x reference)
import functools

import jax
import jax.numpy as jnp
import numpy as np
from jax import lax
from jax.experimental import pallas as pl
from jax.experimental.pallas import tpu as pltpu
from jax.experimental.pallas import tpu_sc as plsc

F = 26
VOC = 1000
D = 16
B = 4096

NC = 2
NS = 16
NW = NC * NS
ROWS_PER_W = B // NW
NG = 8
GROUPS = ROWS_PER_W // NG
XPG = NG * F

NPAIR = (F * (F - 1)) // 2
RPG = NG * 2 * NPAIR
CHUNK = 128
NCH_V = -(-RPG // CHUNK)
RPG_PAD = NCH_V * CHUNK
NCH_W = -(-XPG // CHUNK)
WPG_PAD = NCH_W * CHUNK

_pi = []
_off = []
for _i in range(F):
    for _j in range(_i + 1, F):
        _pi.append(_i)
        _off.append((_i * F + _j) * VOC)
        _pi.append(_j)
        _off.append((_j * F + _i) * VOC)
_pi = np.array(_pi, dtype=np.int32)
_off = np.array(_off, dtype=np.int32)
_q = np.arange(RPG_PAD, dtype=np.int32)
_qm = np.minimum(_q, RPG - 1)
_SRC_V = ((_qm // (2 * NPAIR)) * F + _pi[_qm % (2 * NPAIR)]).astype(np.int32)
_OFF_V = _off[_qm % (2 * NPAIR)].astype(np.int32)
_qw = np.arange(WPG_PAD, dtype=np.int32)
_qwm = np.minimum(_qw, XPG - 1)
_SRC_W = _qwm.astype(np.int32)
_OFF_W = ((_qwm % F) * VOC).astype(np.int32)


def _ffm_sc_body(vflat, w1p, xflat, srcv, offv, srcw, offw, out,
                 idxv_v, rows_v, idxw_v, w1rows_v, out_v, xg_v,
                 srcv_v, offv_v, srcw_v, offw_v, sem):
    wid = lax.axis_index("s") * NC + lax.axis_index("c")
    row0 = wid * ROWS_PER_W

    pltpu.sync_copy(srcv, srcv_v)
    pltpu.sync_copy(offv, offv_v)
    pltpu.sync_copy(srcw, srcw_v)
    pltpu.sync_copy(offw, offw_v)

    def group_body(g, carry):
        xbase = (row0 + g * NG) * F
        pltpu.sync_copy(xflat.at[pl.ds(xbase, XPG)], xg_v)

        for c in range(NCH_V):
            for k in range(CHUNK // D):
                q = c * CHUNK + k * D
                src = srcv_v[pl.ds(q, D)]
                xv = plsc.load_gather(xg_v, [src])
                idxv_v[c, pl.ds(k * D, D)] = xv + offv_v[pl.ds(q, D)]
        for c in range(NCH_W):
            for k in range(CHUNK // D):
                q = c * CHUNK + k * D
                src = srcw_v[pl.ds(q, D)]
                xv = plsc.load_gather(xg_v, [src])
                idxw_v[c, pl.ds(k * D, D)] = xv + offw_v[pl.ds(q, D)]

        copies = []
        for c in range(NCH_V):
            copies.append(
                pltpu.async_copy(
                    vflat.at[idxv_v.at[c]],
                    rows_v.at[pl.ds(c * CHUNK, CHUNK)],
                    sem,
                )
            )
        for c in range(NCH_W):
            copies.append(
                pltpu.async_copy(
                    w1p.at[idxw_v.at[c]],
                    w1rows_v.at[pl.ds(c * CHUNK, CHUNK)],
                    sem,
                )
            )
        for cp in copies:
            cp.wait()

        def row_body(r, acc_out):
            rb = r * (2 * NPAIR)
            wb = r * F
            acc0 = w1rows_v[wb, :]
            acc1 = w1rows_v[wb + 1, :]
            acc2 = w1rows_v[wb + 2, :]
            acc3 = w1rows_v[wb + 3, :]
            for i in range(4, F):
                if i % 4 == 0:
                    acc0 = acc0 + w1rows_v[wb + i, :]
                elif i % 4 == 1:
                    acc1 = acc1 + w1rows_v[wb + i, :]
                elif i % 4 == 2:
                    acc2 = acc2 + w1rows_v[wb + i, :]
                else:
                    acc3 = acc3 + w1rows_v[wb + i, :]
            for p in range(NPAIR):
                prod = rows_v[rb + 2 * p, :] * rows_v[rb + 2 * p + 1, :]
                if p % 4 == 0:
                    acc0 = acc0 + prod
                elif p % 4 == 1:
                    acc1 = acc1 + prod
                elif p % 4 == 2:
                    acc2 = acc2 + prod
                else:
                    acc3 = acc3 + prod
            s = jnp.sum((acc0 + acc1) + (acc2 + acc3))
            lanes = lax.iota(jnp.int32, D)
            return jnp.where(lanes == r, s, acc_out)

        acc_out = lax.fori_loop(0, NG, row_body, jnp.zeros((D,), jnp.float32))
        out_v[pl.ds(g * D, D)] = acc_out
        return carry

    lax.fori_loop(0, GROUPS, group_body, 0)
    pltpu.sync_copy(out_v, out.at[pl.ds(wid * (GROUPS * D), GROUPS * D)])


@functools.cache
def _build_ffm_sc():
    return functools.partial(
        pl.kernel,
        out_type=jax.ShapeDtypeStruct(((B // NG) * D,), jnp.float32),
        mesh=plsc.VectorSubcoreMesh(
            core_axis_name="c", subcore_axis_name="s",
            num_cores=NC, num_subcores=NS),
        scratch_types=[
            pltpu.VMEM((NCH_V, CHUNK), jnp.int32),
            pltpu.VMEM((RPG_PAD, D), jnp.float32),
            pltpu.VMEM((NCH_W, CHUNK), jnp.int32),
            pltpu.VMEM((WPG_PAD, D), jnp.float32),
            pltpu.VMEM((GROUPS * D,), jnp.float32),
            pltpu.VMEM((XPG,), jnp.int32),
            pltpu.VMEM((RPG_PAD,), jnp.int32),
            pltpu.VMEM((RPG_PAD,), jnp.int32),
            pltpu.VMEM((WPG_PAD,), jnp.int32),
            pltpu.VMEM((WPG_PAD,), jnp.int32),
            pltpu.SemaphoreType.DMA,
        ],
        compiler_params=pltpu.CompilerParams(
            needs_layout_passes=False, use_tc_tiling_on_sc=False),
    )(_ffm_sc_body)


def kernel(x, W1, V, bias):
    xflat = x.astype(jnp.int32).reshape(B * F)
    vflat = V.reshape(F * F * VOC, D)
    w1p = jnp.pad(W1.reshape(F * VOC, 1), ((0, 0), (0, D - 1)))

    out = _build_ffm_sc()(
        vflat, w1p, xflat,
        jnp.asarray(_SRC_V), jnp.asarray(_OFF_V),
        jnp.asarray(_SRC_W), jnp.asarray(_OFF_W),
    )
    out = out.reshape(B // NG, D)[:, :NG].reshape(B, 1)
    return out + bias


# speedup vs baseline: 2.9894x; 2.9894x over previous
import functools

import jax
import jax.numpy as jnp
from jax import lax
from jax.experimental import pallas as pl
from jax.experimental.pallas import tpu as pltpu
from jax.experimental.pallas import tpu_sc as plsc

F = 26
VOC = 1000
D = 16
B = 4096

NC = 2
NS = 16
NW = NC * NS
NPAIR = (F * (F - 1)) // 2
PAIRS_PER_W = -(-NPAIR // NW)
NBC = B // D
SEG = B // NS


def _decode_pair(p):
    def body(t, carry):
        rem, ii, act = carry
        rowlen = F - 1 - t
        take = jnp.logical_and(act == 1, rem >= rowlen)
        rem = jnp.where(take, rem - rowlen, rem)
        ii = jnp.where(take, ii + 1, ii)
        act = jnp.where(take, act, 0)
        return (rem, ii, act)

    rem, ii, _ = lax.fori_loop(0, F - 1, body, (p, 0, 1))
    return ii, ii + 1 + rem


def _ffm_sc_body(vd, w1d, xt, out,
                 tbla_v, tblb_v, cola_v, colb_v, partial_v,
                 w1line_v, seg_v, segtmp_v, shared, sem):
    scid = lax.axis_index("c")
    sid = lax.axis_index("s")
    wid = sid * NC + scid

    zero16 = jnp.zeros((D,), jnp.float32)

    def zero_body(c, carry):
        partial_v[pl.ds(c * D, D)] = zero16
        return carry

    lax.fori_loop(0, B // D, zero_body, 0)

    @pl.when(wid < F)
    def _():
        f = wid
        pltpu.sync_copy(w1d.at[f], w1line_v)
        pltpu.sync_copy(xt.at[f], cola_v)

        def fo_body(bc, carry):
            xi = cola_v[pl.ds(bc * D, D)]
            w = plsc.load_gather(w1line_v, [xi])
            plsc.addupdate(partial_v.at[pl.ds(bc * D, D)], w)
            return carry

        lax.fori_loop(0, NBC, fo_body, 0)

    for k in range(PAIRS_PER_W):
        p = wid + NW * k
        valid = p < NPAIR
        pc = jnp.minimum(p, NPAIR - 1)
        i, j = _decode_pair(pc)

        @pl.when(valid)
        def _():
            pltpu.sync_copy(vd.at[pl.ds((i * F + j) * D, D)], tbla_v)
            pltpu.sync_copy(vd.at[pl.ds((j * F + i) * D, D)], tblb_v)
            pltpu.sync_copy(xt.at[i], cola_v)
            pltpu.sync_copy(xt.at[j], colb_v)

            def pair_body(bc, carry):
                xi = cola_v[pl.ds(bc * D, D)]
                xj = colb_v[pl.ds(bc * D, D)]
                acc0 = zero16
                acc1 = zero16
                for d in range(D):
                    dsplat = jnp.full((D,), d, jnp.int32)
                    a = plsc.load_gather(tbla_v, [dsplat, xi])
                    b = plsc.load_gather(tblb_v, [dsplat, xj])
                    if d % 2 == 0:
                        acc0 = acc0 + a * b
                    else:
                        acc1 = acc1 + a * b
                plsc.addupdate(partial_v.at[pl.ds(bc * D, D)], acc0 + acc1)
                return carry

            lax.fori_loop(0, NBC, pair_body, 0)

    pltpu.sync_copy(partial_v, shared.at[sid])
    plsc.subcore_barrier()

    def zseg_body(c, carry):
        seg_v[pl.ds(c * D, D)] = zero16
        return carry

    lax.fori_loop(0, SEG // D, zseg_body, 0)

    def red_body(t, carry):
        pltpu.sync_copy(shared.at[t, pl.ds(sid * SEG, SEG)], segtmp_v)
        for c in range(SEG // D):
            sl = pl.ds(c * D, D)
            seg_v[sl] = seg_v[sl] + segtmp_v[sl]
        return carry

    lax.fori_loop(0, NS, red_body, 0)
    pltpu.sync_copy(seg_v, out.at[pl.ds(scid * B + sid * SEG, SEG)])


@functools.cache
def _build_ffm_sc():
    return functools.partial(
        pl.kernel,
        out_type=jax.ShapeDtypeStruct((NC * B,), jnp.float32),
        mesh=plsc.VectorSubcoreMesh(
            core_axis_name="c", subcore_axis_name="s",
            num_cores=NC, num_subcores=NS),
        scratch_types=[
            pltpu.VMEM((D, VOC), jnp.float32),
            pltpu.VMEM((D, VOC), jnp.float32),
            pltpu.VMEM((B,), jnp.int32),
            pltpu.VMEM((B,), jnp.int32),
            pltpu.VMEM((B,), jnp.float32),
            pltpu.VMEM((VOC,), jnp.float32),
            pltpu.VMEM((SEG,), jnp.float32),
            pltpu.VMEM((SEG,), jnp.float32),
            pltpu.VMEM_SHARED((NS, B), jnp.float32),
            pltpu.SemaphoreType.DMA,
        ],
        compiler_params=pltpu.CompilerParams(
            needs_layout_passes=False, use_tc_tiling_on_sc=False),
    )(_ffm_sc_body)


def kernel(x, W1, V, bias):
    vd = jnp.swapaxes(V, 2, 3).reshape(F * F * D, VOC)
    w1d = jnp.swapaxes(W1, 1, 2).reshape(F, VOC)
    xt = x.astype(jnp.int32).T

    out = _build_ffm_sc()(vd, w1d, xt)
    return (out[:B] + out[B:]).reshape(B, 1) + bias


# speedup vs baseline: 3.4563x; 1.1562x over previous
import functools

import jax
import jax.numpy as jnp
from jax import lax
from jax.experimental import pallas as pl
from jax.experimental.pallas import tpu as pltpu
from jax.experimental.pallas import tpu_sc as plsc

F = 26
VOC = 1000
D = 16
B = 4096

NC = 2
NS = 16
NW = NC * NS
NPAIR = (F * (F - 1)) // 2
PAIRS_PER_W = -(-NPAIR // NW)
NBC = B // D
SEG = B // NS
XW = SEG * F


def _decode_pair(p):
    def body(t, carry):
        rem, ii, act = carry
        rowlen = F - 1 - t
        take = jnp.logical_and(act == 1, rem >= rowlen)
        rem = jnp.where(take, rem - rowlen, rem)
        ii = jnp.where(take, ii + 1, ii)
        act = jnp.where(take, act, 0)
        return (rem, ii, act)

    rem, ii, _ = lax.fori_loop(0, F - 1, body, (p, 0, 1))
    return ii, ii + 1 + rem


def _ffm_sc_body(vd, w1d, xflat, out,
                 tbla0, tblb0, tbla1, tblb1, cola_v, colb_v, partial_v,
                 w1line_v, seg_v, segtmp_v, xwin_v, coltile_v,
                 xtsh, shared, sem0, sem1):
    scid = lax.axis_index("c")
    sid = lax.axis_index("s")
    wid = sid * NC + scid
    sems = (sem0, sem1)
    tbls = ((tbla0, tblb0), (tbla1, tblb1))

    pltpu.sync_copy(xflat.at[pl.ds(sid * XW, XW)], xwin_v)
    lanes = lax.iota(jnp.int32, D)
    lanesF = lanes * F
    for i in range(F):
        for c in range(SEG // D):
            idx16 = lanesF + (c * D * F + i)
            v = plsc.load_gather(xwin_v, [idx16])
            coltile_v[i, pl.ds(c * D, D)] = v
    pltpu.sync_copy(coltile_v, xtsh.at[:, pl.ds(sid * SEG, SEG)])

    zero16 = jnp.zeros((D,), jnp.float32)

    def zero_body(c, carry):
        partial_v[pl.ds(c * D, D)] = zero16
        return carry

    lax.fori_loop(0, B // D, zero_body, 0)
    plsc.subcore_barrier()

    @pl.when(wid < F)
    def _():
        f = wid
        pltpu.sync_copy(w1d.at[f], w1line_v)
        pltpu.sync_copy(xtsh.at[f], cola_v)

        def fo_body(bc, carry):
            xi = cola_v[pl.ds(bc * D, D)]
            w = plsc.load_gather(w1line_v, [xi])
            plsc.addupdate(partial_v.at[pl.ds(bc * D, D)], w)
            return carry

        lax.fori_loop(0, NBC, fo_body, 0)

    def start_fetch(k):
        p = wid + NW * k
        valid = p < NPAIR
        pc = jnp.minimum(p, NPAIR - 1)
        i, j = _decode_pair(pc)
        ta, tb = tbls[k % 2]
        cps = [
            pltpu.async_copy(
                vd.at[pl.ds((i * F + j) * D, D)], ta, sems[k % 2]),
            pltpu.async_copy(
                vd.at[pl.ds((j * F + i) * D, D)], tb, sems[k % 2]),
        ]
        return (i, j, valid, cps)

    inflight = start_fetch(0)
    for k in range(PAIRS_PER_W):
        i, j, valid, cps = inflight
        nxt = start_fetch(k + 1) if k + 1 < PAIRS_PER_W else None
        ta, tb = tbls[k % 2]

        for cp in cps:
            cp.wait()

        @pl.when(valid)
        def _():
            pltpu.sync_copy(xtsh.at[i], cola_v)
            pltpu.sync_copy(xtsh.at[j], colb_v)

            def pair_body(bc, carry):
                xi = cola_v[pl.ds(bc * D, D)]
                xj = colb_v[pl.ds(bc * D, D)]
                acc0 = zero16
                acc1 = zero16
                for d in range(D):
                    dsplat = jnp.full((D,), d, jnp.int32)
                    a = plsc.load_gather(ta, [dsplat, xi])
                    b = plsc.load_gather(tb, [dsplat, xj])
                    if d % 2 == 0:
                        acc0 = acc0 + a * b
                    else:
                        acc1 = acc1 + a * b
                plsc.addupdate(partial_v.at[pl.ds(bc * D, D)], acc0 + acc1)
                return carry

            lax.fori_loop(0, NBC, pair_body, 0)

        inflight = nxt

    pltpu.sync_copy(partial_v, shared.at[sid])
    plsc.subcore_barrier()

    def zseg_body(c, carry):
        seg_v[pl.ds(c * D, D)] = zero16
        return carry

    lax.fori_loop(0, SEG // D, zseg_body, 0)

    def red_body(t, carry):
        pltpu.sync_copy(shared.at[t, pl.ds(sid * SEG, SEG)], segtmp_v)
        for c in range(SEG // D):
            sl = pl.ds(c * D, D)
            seg_v[sl] = seg_v[sl] + segtmp_v[sl]
        return carry

    lax.fori_loop(0, NS, red_body, 0)
    pltpu.sync_copy(seg_v, out.at[pl.ds(scid * B + sid * SEG, SEG)])


@functools.cache
def _build_ffm_sc():
    return functools.partial(
        pl.kernel,
        out_type=jax.ShapeDtypeStruct((NC * B,), jnp.float32),
        mesh=plsc.VectorSubcoreMesh(
            core_axis_name="c", subcore_axis_name="s",
            num_cores=NC, num_subcores=NS),
        scratch_types=[
            pltpu.VMEM((D, VOC), jnp.float32),
            pltpu.VMEM((D, VOC), jnp.float32),
            pltpu.VMEM((D, VOC), jnp.float32),
            pltpu.VMEM((D, VOC), jnp.float32),
            pltpu.VMEM((B,), jnp.int32),
            pltpu.VMEM((B,), jnp.int32),
            pltpu.VMEM((B,), jnp.float32),
            pltpu.VMEM((VOC,), jnp.float32),
            pltpu.VMEM((SEG,), jnp.float32),
            pltpu.VMEM((SEG,), jnp.float32),
            pltpu.VMEM((XW,), jnp.int32),
            pltpu.VMEM((F, SEG), jnp.int32),
            pltpu.VMEM_SHARED((F, B), jnp.int32),
            pltpu.VMEM_SHARED((NS, B), jnp.float32),
            pltpu.SemaphoreType.DMA,
            pltpu.SemaphoreType.DMA,
        ],
        compiler_params=pltpu.CompilerParams(
            needs_layout_passes=False, use_tc_tiling_on_sc=False),
    )(_ffm_sc_body)


def kernel(x, W1, V, bias):
    vd = jnp.swapaxes(V, 2, 3).reshape(F * F * D, VOC)
    w1d = jnp.swapaxes(W1, 1, 2).reshape(F, VOC)
    xflat = x.astype(jnp.int32).reshape(B * F)

    out = _build_ffm_sc()(vd, w1d, xflat)
    return (out[:B] + out[B:]).reshape(B, 1) + bias


# speedup vs baseline: 4.0201x; 1.1631x over previous
import functools

import jax
import jax.numpy as jnp
from jax import lax
from jax.experimental import pallas as pl
from jax.experimental.pallas import tpu as pltpu
from jax.experimental.pallas import tpu_sc as plsc

F = 26
VOC = 1000
D = 16
B = 4096

NC = 2
NS = 16
NW = NC * NS
NPAIR = (F * (F - 1)) // 2
PAIRS_PER_W = -(-NPAIR // NW)
NBC = B // D
SEG = B // NS
XW = SEG * F


def _decode_pair(p):
    def body(t, carry):
        rem, ii, act = carry
        rowlen = F - 1 - t
        take = jnp.logical_and(act == 1, rem >= rowlen)
        rem = jnp.where(take, rem - rowlen, rem)
        ii = jnp.where(take, ii + 1, ii)
        act = jnp.where(take, act, 0)
        return (rem, ii, act)

    rem, ii, _ = lax.fori_loop(0, F - 1, body, (p, 0, 1))
    return ii, ii + 1 + rem


def _ffm_sc_body(vd, w1d, xflat, out,
                 tbla0, tblb0, tbla1, tblb1, cola_v, colb_v, partial_v,
                 w1line_v, seg_v, segtmp_v, xwin_v, coltile_v,
                 xtsh, shared, sem0, sem1):
    scid = lax.axis_index("c")
    sid = lax.axis_index("s")
    wid = sid * NC + scid
    sems = (sem0, sem1)
    tbls = ((tbla0, tblb0), (tbla1, tblb1))

    pltpu.sync_copy(xflat.at[pl.ds(sid * XW, XW)], xwin_v)
    lanes = lax.iota(jnp.int32, D)
    lanesF = lanes * F
    for i in range(F):
        for c in range(SEG // D):
            idx16 = lanesF + (c * D * F + i)
            v = plsc.load_gather(xwin_v, [idx16])
            coltile_v[i, pl.ds(c * D, D)] = v
    pltpu.sync_copy(coltile_v, xtsh.at[:, pl.ds(sid * SEG, SEG)])

    zero16 = jnp.zeros((D,), jnp.float32)

    def zero_body(c, carry):
        partial_v[pl.ds(c * D, D)] = zero16
        return carry

    lax.fori_loop(0, B // D, zero_body, 0)
    plsc.subcore_barrier()

    @pl.when(wid < F)
    def _():
        f = wid
        pltpu.sync_copy(w1d.at[f], w1line_v)
        pltpu.sync_copy(xtsh.at[f], cola_v)

        def fo_body(bc, carry):
            xi = cola_v[pl.ds(bc * D, D)]
            w = plsc.load_gather(w1line_v, [xi])
            plsc.addupdate(partial_v.at[pl.ds(bc * D, D)], w)
            return carry

        lax.fori_loop(0, NBC, fo_body, 0)

    def start_fetch(k):
        p = wid + NW * k
        valid = p < NPAIR
        pc = jnp.minimum(p, NPAIR - 1)
        i, j = _decode_pair(pc)
        ta, tb = tbls[k % 2]
        cps = [
            pltpu.async_copy(
                vd.at[pl.ds((i * F + j) * D, D)], ta, sems[k % 2]),
            pltpu.async_copy(
                vd.at[pl.ds((j * F + i) * D, D)], tb, sems[k % 2]),
        ]
        return (i, j, valid, cps)

    inflight = start_fetch(0)
    for k in range(PAIRS_PER_W):
        i, j, valid, cps = inflight
        nxt = start_fetch(k + 1) if k + 1 < PAIRS_PER_W else None
        ta, tb = tbls[k % 2]

        for cp in cps:
            cp.wait()

        @pl.when(valid)
        def _():
            pltpu.sync_copy(xtsh.at[i], cola_v)
            pltpu.sync_copy(xtsh.at[j], colb_v)

            def pair_body(bc, carry):
                xi = cola_v[pl.ds(bc * D, D)]
                xj = colb_v[pl.ds(bc * D, D)]
                acc0 = zero16
                acc1 = zero16
                for d in range(D):
                    dsplat = jnp.full((D,), d, jnp.int32)
                    a = plsc.load_gather(ta, [dsplat, xi])
                    b = plsc.load_gather(tb, [dsplat, xj])
                    if d % 2 == 0:
                        acc0 = acc0 + a * b
                    else:
                        acc1 = acc1 + a * b
                plsc.addupdate(partial_v.at[pl.ds(bc * D, D)], acc0 + acc1)
                return carry

            lax.fori_loop(0, NBC, pair_body, 0)

        inflight = nxt

    pltpu.sync_copy(partial_v, shared.at[sid])
    plsc.subcore_barrier()

    def zseg_body(c, carry):
        seg_v[pl.ds(c * D, D)] = zero16
        return carry

    lax.fori_loop(0, SEG // D, zseg_body, 0)

    def red_body(t, carry):
        pltpu.sync_copy(shared.at[t, pl.ds(sid * SEG, SEG)], segtmp_v)
        for c in range(SEG // D):
            sl = pl.ds(c * D, D)
            seg_v[sl] = seg_v[sl] + segtmp_v[sl]
        return carry

    lax.fori_loop(0, NS, red_body, 0)
    pltpu.sync_copy(seg_v, out.at[pl.ds(scid * B + sid * SEG, SEG)])


@functools.cache
def _build_ffm_sc():
    return functools.partial(
        pl.kernel,
        out_type=jax.ShapeDtypeStruct((NC * B,), jnp.float32),
        mesh=plsc.VectorSubcoreMesh(
            core_axis_name="c", subcore_axis_name="s",
            num_cores=NC, num_subcores=NS),
        scratch_types=[
            pltpu.VMEM((D, VOC), jnp.float32),
            pltpu.VMEM((D, VOC), jnp.float32),
            pltpu.VMEM((D, VOC), jnp.float32),
            pltpu.VMEM((D, VOC), jnp.float32),
            pltpu.VMEM((B,), jnp.int32),
            pltpu.VMEM((B,), jnp.int32),
            pltpu.VMEM((B,), jnp.float32),
            pltpu.VMEM((VOC,), jnp.float32),
            pltpu.VMEM((SEG,), jnp.float32),
            pltpu.VMEM((SEG,), jnp.float32),
            pltpu.VMEM((XW,), jnp.int32),
            pltpu.VMEM((F, SEG), jnp.int32),
            pltpu.VMEM_SHARED((F, B), jnp.int32),
            pltpu.VMEM_SHARED((NS, B), jnp.float32),
            pltpu.SemaphoreType.DMA,
            pltpu.SemaphoreType.DMA,
        ],
        compiler_params=pltpu.CompilerParams(
            needs_layout_passes=False, use_tc_tiling_on_sc=True),
    )(_ffm_sc_body)


def kernel(x, W1, V, bias):
    vd = jnp.swapaxes(V, 2, 3).reshape(F * F * D, VOC)
    w1d = jnp.swapaxes(W1, 1, 2).reshape(F, VOC)
    xflat = x.astype(jnp.int32).reshape(B * F)

    out = _build_ffm_sc()(vd, w1d, xflat)
    return (out[:B] + out[B:]).reshape(B, 1) + bias
